# baseline (speedup 1.0000x reference)
import functools

import jax
import jax.numpy as jnp
from jax import lax
from jax.experimental import pallas as pl
from jax.experimental.pallas import tpu as pltpu
from jax.experimental.pallas import tpu_sc as plsc

_NC = 2
_NS = 16
_NW = _NC * _NS
_K = 120
_NBUF_G = 4
_NBUF_S = 3
_T = 8
_G = 256


def _mid_body(hf_ref, hs_ref, w1_ref, bm1_ref, w2_ref, bm2_ref, m_ref):
    u = jnp.concatenate([hf_ref[...], hs_ref[...]], axis=1)
    m1 = jnp.maximum(
        jnp.dot(u, w1_ref[...], preferred_element_type=jnp.float32)
        + bm1_ref[...], 0.0)
    m_ref[...] = jnp.maximum(
        jnp.dot(m1, w2_ref[...], preferred_element_type=jnp.float32)
        + bm2_ref[...], 0.0)


def _gru_body(s_ref, h_ref, gk_ref, grk_ref, gb0_ref, gb1_ref, ho_ref):
    x = s_ref[...]
    h = h_ref[...]
    mx = jnp.dot(x, gk_ref[...], preferred_element_type=jnp.float32) + gb0_ref[...]
    mh = jnp.dot(h, grk_ref[...], preferred_element_type=jnp.float32) + gb1_ref[...]
    d = h.shape[1]
    xz, xr, xh = mx[:, :d], mx[:, d:2 * d], mx[:, 2 * d:]
    rz, rr, rh = mh[:, :d], mh[:, d:2 * d], mh[:, 2 * d:]
    z = jax.nn.sigmoid(xz + rz)
    r = jax.nn.sigmoid(xr + rr)
    hh = jnp.tanh(xh + r * rh)
    ho_ref[...] = z * h + (1.0 - z) * hh


def _pool_body(ids_ref, h_ref, wr1_ref, br1_ref, wr2_ref, br2_ref,
               out_ref, acc_ref):
    i = pl.program_id(0)

    @pl.when(i == 0)
    def _():
        acc_ref[...] = jnp.zeros_like(acc_ref)

    ids = ids_ref[0]
    bn = ids.shape[1]
    oh = (lax.broadcasted_iota(jnp.int32, (_G, bn), 0)
          == jnp.broadcast_to(ids, (_G, bn))).astype(jnp.float32)
    acc_ref[...] += jnp.dot(oh, h_ref[...], preferred_element_type=jnp.float32,
                            precision=lax.Precision.HIGHEST)

    @pl.when(i == pl.num_programs(0) - 1)
    def _():
        p = acc_ref[...]
        r = jnp.maximum(
            jnp.dot(p, wr1_ref[...], preferred_element_type=jnp.float32)
            + br1_ref[...], 0.0)
        out_ref[...] = (jnp.dot(r, wr2_ref[...],
                                preferred_element_type=jnp.float32)
                        + br2_ref[...])


def _lane0(v):
    lane = lax.iota(jnp.int32, 16)
    return lax.reduce_sum_p.bind(
        jnp.where(lane == 0, v, jnp.zeros_like(v)), axes=(0,))


def _sc_gather_body(h_hbm, f_hbm, sg_hbm, hf_hbm, hs_hbm, *scr):
    nbuf = _NBUF_G
    ia = scr[0:nbuf]
    ib = scr[nbuf:2 * nbuf]
    ra = scr[2 * nbuf:3 * nbuf]
    rb = scr[3 * nbuf:4 * nbuf]
    semi = scr[4 * nbuf]
    semg = scr[4 * nbuf + 1:5 * nbuf + 1]
    semw = scr[5 * nbuf + 1:6 * nbuf + 1]
    cid = lax.axis_index("c")
    sid = lax.axis_index("s")
    wid = sid * _NC + cid
    nout = (f_hbm.shape[0] // _K) // (_NW * nbuf)

    def cbase(g, b):
        return ((g * nbuf + b) * _NW + wid) * _K

    def fire_idx(g):
        for b in range(nbuf):
            base = cbase(g, b)
            pltpu.async_copy(f_hbm.at[pl.ds(base, _K)], ia[b], semi)
            pltpu.async_copy(sg_hbm.at[pl.ds(base, _K)], ib[b], semi)

    fire_idx(0)

    def outer(g, carry):
        for b in range(nbuf):
            base = cbase(g, b)
            pltpu.make_async_copy(f_hbm.at[pl.ds(base, _K)], ia[b], semi).wait()
            pltpu.make_async_copy(sg_hbm.at[pl.ds(base, _K)], ib[b], semi).wait()
        for b in range(nbuf):
            @pl.when(g > 0)
            def _(b=b):
                prev = cbase(g - 1, b)
                pltpu.make_async_copy(
                    ra[b], hf_hbm.at[pl.ds(prev, _K), :], semw[b]).wait()
                pltpu.make_async_copy(
                    rb[b], hs_hbm.at[pl.ds(prev, _K), :], semw[b]).wait()
            pltpu.async_copy(h_hbm.at[ia[b]], ra[b], semg[b])
            pltpu.async_copy(h_hbm.at[ib[b]], rb[b], semg[b])

        for b in range(nbuf):
            base = cbase(g, b)
            pltpu.make_async_copy(h_hbm.at[ia[b]], ra[b], semg[b]).wait()
            pltpu.make_async_copy(h_hbm.at[ib[b]], rb[b], semg[b]).wait()
            @pl.when(g < nout - 1)
            def _(b=b):
                nbase = cbase(g + 1, b)
                pltpu.async_copy(f_hbm.at[pl.ds(nbase, _K)], ia[b], semi)
                pltpu.async_copy(sg_hbm.at[pl.ds(nbase, _K)], ib[b], semi)
            pltpu.async_copy(ra[b], hf_hbm.at[pl.ds(base, _K), :], semw[b])
            pltpu.async_copy(rb[b], hs_hbm.at[pl.ds(base, _K), :], semw[b])
        return carry

    lax.fori_loop(0, nout, outer, 0)
    for b in range(nbuf):
        last = cbase(nout - 1, b)
        pltpu.make_async_copy(ra[b], hf_hbm.at[pl.ds(last, _K), :],
                              semw[b]).wait()
        pltpu.make_async_copy(rb[b], hs_hbm.at[pl.ds(last, _K), :],
                              semw[b]).wait()


def _sc_scatter_body(m_hbm, ss_hbm, est_hbm, out_hbm, *scr):
    ib = scr[0:2]
    mb = scr[2:4]
    estv = scr[4]
    acc = scr[5]
    semc = scr[6:8]
    cid = lax.axis_index("c")
    sid = lax.axis_index("s")
    w = cid * _NS + sid
    rows_pt = acc.shape[0] - 8
    d = acc.shape[1]
    nvec = d // 16

    def zbody(t, c):
        acc[t // nvec, pl.ds((t % nvec) * 16, 16)] = jnp.zeros((16,), jnp.float32)
        return c
    lax.fori_loop(0, acc.shape[0] * nvec, zbody, 0, unroll=8)

    pltpu.sync_copy(est_hbm, estv)
    e0 = _lane0(estv[pl.ds(w, 16)])
    e1 = _lane0(estv[pl.ds(w + 1, 16)])
    base8 = (e0 // 8) * 8
    nch = (e1 - base8 + _K - 1) // _K
    vbase = w * rows_pt

    def fire(ci, b):
        cb = base8 + ci * _K
        pltpu.async_copy(ss_hbm.at[pl.ds(cb, _K)], ib[b].at[pl.ds(0, _K)],
                         semc[b])
        pltpu.async_copy(m_hbm.at[pl.ds(cb, _K), :], mb[b], semc[b])

    def waitcp(ci, b):
        cb = base8 + ci * _K
        pltpu.make_async_copy(ss_hbm.at[pl.ds(cb, _K)],
                              ib[b].at[pl.ds(0, _K)], semc[b]).wait()
        pltpu.make_async_copy(m_hbm.at[pl.ds(cb, _K), :], mb[b],
                              semc[b]).wait()

    def rows(b):
        def row(r, c2):
            dst = _lane0(ib[b][pl.ds(r, 16)])
            dl = dst - vbase
            ok = jnp.logical_and(dl >= 0, dl < rows_pt)
            dl2 = jnp.where(ok, dl, rows_pt)
            for cc in range(nvec):
                sl = pl.ds(cc * 16, 16)
                acc[dl2, sl] = acc[dl2, sl] + mb[b][r, sl]
            return c2
        lax.fori_loop(0, _K, row, 0)

    @pl.when(nch > 0)
    def _():
        fire(0, 0)

        def pair(p, carry):
            ci0 = 2 * p
            ci1 = 2 * p + 1

            @pl.when(ci1 < nch)
            def _():
                fire(ci1, 1)
            waitcp(ci0, 0)
            rows(0)

            @pl.when(ci1 < nch)
            def _():
                @pl.when(ci1 + 1 < nch)
                def _():
                    fire(ci1 + 1, 0)
                waitcp(ci1, 1)
                rows(1)
            return carry

        lax.fori_loop(0, (nch + 1) // 2, pair, 0)

    pltpu.sync_copy(acc.at[pl.ds(0, rows_pt), :],
                    out_hbm.at[pl.ds(vbase, rows_pt), :])


def kernel(link_state, states_first, states_second, states_graph_ids,
           sates_num_edges, Wm1, bm1, Wm2, bm2, gru_k, gru_rk, gru_b,
           Wr1, br1, Wr2, br2):
    n, d = link_state.shape
    e = states_first.shape[0]
    f32 = jnp.float32

    bm1r = bm1.reshape(1, d)
    bm2r = bm2.reshape(1, d)
    gb0 = gru_b[0:1]
    gb1 = gru_b[1:2]
    br1r = br1.reshape(1, d)
    br2r = br2.reshape(1, 1)

    n_pad = ((n + 8 * _NW - 1) // (8 * _NW)) * (8 * _NW)
    rows_pt = n_pad // _NW

    grain = _K * _NW * _NBUF_G * _NBUF_S
    e_pad = ((e + grain - 1) // grain) * grain
    pad = e_pad - e
    zpad = jnp.zeros((pad,), jnp.int32)
    perm = jnp.argsort(states_second, stable=True)
    f_sorted = states_first.astype(jnp.int32)[perm]
    s_sorted = states_second.astype(jnp.int32)[perm]
    f_pad = jnp.concatenate([f_sorted, zpad])
    sg_pad = jnp.concatenate([s_sorted, zpad])
    ss_pad = jnp.concatenate([s_sorted, jnp.full((pad,), n, jnp.int32)])
    est = jnp.searchsorted(s_sorted,
                           jnp.arange(_NW + 1, dtype=jnp.int32) * rows_pt
                           ).astype(jnp.int32)
    est48 = jnp.zeros((48,), jnp.int32).at[:_NW + 1].set(est)

    bn = 2000
    nb = n // bn
    be = 2560
    neb = e_pad // be
    assert e_pad % be == 0

    wspec = pl.BlockSpec((d, d), lambda i: (0, 0))
    w3spec = pl.BlockSpec((d, 3 * d), lambda i: (0, 0))
    bspec = pl.BlockSpec((1, d), lambda i: (0, 0))
    b3spec = pl.BlockSpec((1, 3 * d), lambda i: (0, 0))
    espec = pl.BlockSpec((be, d), lambda i: (i, 0))

    mid_call = pl.pallas_call(
        _mid_body,
        grid=(neb,),
        in_specs=[espec, espec,
                  pl.BlockSpec((2 * d, d), lambda i: (0, 0)), bspec,
                  wspec, bspec],
        out_specs=espec,
        out_shape=jax.ShapeDtypeStruct((e_pad, d), f32),
    )

    gru_call = pl.pallas_call(
        _gru_body,
        grid=(nb,),
        in_specs=[pl.BlockSpec((bn, d), lambda i: (i, 0)),
                  pl.BlockSpec((bn, d), lambda i: (i, 0)),
                  w3spec, w3spec, b3spec, b3spec],
        out_specs=pl.BlockSpec((bn, d), lambda i: (i, 0)),
        out_shape=jax.ShapeDtypeStruct((n, d), f32),
    )

    pool_call = pl.pallas_call(
        _pool_body,
        grid=(nb,),
        in_specs=[pl.BlockSpec((1, 1, bn), lambda i: (i, 0, 0)),
                  pl.BlockSpec((bn, d), lambda i: (i, 0)),
                  wspec, bspec, pl.BlockSpec((d, 1), lambda i: (0, 0)),
                  pl.BlockSpec((1, 1), lambda i: (0, 0))],
        out_specs=pl.BlockSpec((_G, 1), lambda i: (0, 0)),
        out_shape=jax.ShapeDtypeStruct((_G, 1), f32),
        scratch_shapes=[pltpu.VMEM((_G, d), f32)],
        compiler_params=pltpu.CompilerParams(
            dimension_semantics=("arbitrary",)),
    )

    mesh = plsc.VectorSubcoreMesh(core_axis_name="c", subcore_axis_name="s")

    gather_call = functools.partial(
        pl.kernel, mesh=mesh,
        out_type=[jax.ShapeDtypeStruct((e_pad, d), f32)] * 2,
        scratch_types=(
            [pltpu.VMEM((_K,), jnp.int32)] * (2 * _NBUF_G)
            + [pltpu.VMEM((_K, d), f32)] * (2 * _NBUF_G)
            + [pltpu.SemaphoreType.DMA] * (2 * _NBUF_G + 1)),
    )(_sc_gather_body)

    scatter_call = functools.partial(
        pl.kernel, mesh=mesh,
        out_type=jax.ShapeDtypeStruct((n_pad, d), f32),
        scratch_types=(
            [pltpu.VMEM((_K + 16,), jnp.int32)] * 2
            + [pltpu.VMEM((_K, d), f32)] * 2
            + [pltpu.VMEM((48,), jnp.int32)]
            + [pltpu.VMEM((n_pad // _NW + 8, d), f32)]
            + [pltpu.SemaphoreType.DMA] * 2),
        compiler_params=pltpu.CompilerParams(needs_layout_passes=False),
    )(_sc_scatter_body)

    ids3 = states_graph_ids.astype(jnp.int32).reshape(nb, 1, bn)

    h = link_state
    for _ in range(_T):
        hf, hs = gather_call(h, f_pad, sg_pad)
        m = mid_call(hf, hs, Wm1, bm1r, Wm2, bm2r)
        s = scatter_call(m, ss_pad, est48)
        h = gru_call(s, h, gru_k, gru_rk, gb0, gb1)
    out = pool_call(ids3, h, Wr1, br1r, Wr2, br2r)
    return out
